# Initial kernel scaffold; baseline (speedup 1.0000x reference)
import jax
import jax.numpy as jnp
from jax.experimental import pallas as pl


def kernel(x, class_ids, W, b):
    raise NotImplementedError("write your pallas kernel here")


# speedup vs baseline: 3.2817x; 3.2817x over previous
import jax
import jax.numpy as jnp
from jax.experimental import pallas as pl
from jax.experimental.pallas import tpu as pltpu

E = 8
BLK = 256


def _dispatch_block(ids_ref, x_ref, W_ref, b_ref, o_ref):
    ids = ids_ref[0, 0, :]
    x = x_ref[...]
    acc = jnp.zeros(o_ref.shape, dtype=jnp.float32)
    for e in range(E):
        m = (ids == e).astype(jnp.float32)
        xm = x * m[:, None]
        acc = acc + jax.lax.dot_general(
            xm, W_ref[e],
            (((1,), (1,)), ((), ())),
            preferred_element_type=jnp.float32,
        )
        acc = acc + m[:, None] * b_ref[e][None, :]
    o_ref[...] = acc


def kernel(x, class_ids, W, b):
    N, D_IN = x.shape
    _, D_OUT, _ = W.shape
    ids = class_ids.astype(jnp.int32).reshape(N // BLK, 1, BLK)
    grid = (N // BLK,)
    out = pl.pallas_call(
        _dispatch_block,
        grid=grid,
        in_specs=[
            pl.BlockSpec((1, 1, BLK), lambda i: (i, 0, 0)),
            pl.BlockSpec((BLK, D_IN), lambda i: (i, 0)),
            pl.BlockSpec((E, D_OUT, D_IN), lambda i: (0, 0, 0)),
            pl.BlockSpec((E, D_OUT), lambda i: (0, 0)),
        ],
        out_specs=pl.BlockSpec((BLK, D_OUT), lambda i: (i, 0)),
        out_shape=jax.ShapeDtypeStruct((N, D_OUT), jnp.float32),
    )(ids, x, W, b)
    return out
